# Initial kernel scaffold; baseline (speedup 1.0000x reference)
import jax
import jax.numpy as jnp
from jax.experimental import pallas as pl


def kernel(x, topk_attn, topk_idx, W, b, A_pool, B_pool, bias_pool):
    raise NotImplementedError("write your pallas kernel here")


# speedup vs baseline: 7.2372x; 7.2372x over previous
import jax
import jax.numpy as jnp
from jax.experimental import pallas as pl
from jax.experimental.pallas import tpu as pltpu

_BSZ, _SEQ, _DIN, _DOUT, _E, _K, _R = 4, 2048, 768, 768, 64, 8, 16


def _moelora_body(idx_ref, attn_ref, x_ref, wt_ref, b_ref, apt_ref, bp_ref,
                  bias_ref, out_ref):
    bi = pl.program_id(0)
    a_parts = []
    b_parts = []
    bias_acc = b_ref[:]
    for k in range(_K):
        e = idx_ref[bi, k]
        w = attn_ref[bi, k]
        a_parts.append(apt_ref[pl.ds(e, 1)].reshape(_R, _DIN))
        b_parts.append(bp_ref[pl.ds(e, 1)].reshape(_R, _DOUT) * w)
        bias_acc = bias_acc + w * bias_ref[pl.ds(e, 1), :]
    acat_t = jnp.concatenate(a_parts, axis=0)
    bcat = jnp.concatenate(b_parts, axis=0)
    delta = jax.lax.dot_general(
        acat_t, bcat, (((0,), (0,)), ((), ())),
        preferred_element_type=jnp.float32)
    m = wt_ref[:] + delta
    out_ref[0] = jnp.dot(x_ref[0], m,
                         preferred_element_type=jnp.float32) + bias_acc


@jax.jit
def _run(x, attn, idx, wt, b2, apt, bp, bias_pool):
    return pl.pallas_call(
        _moelora_body,
        grid=(_BSZ,),
        in_specs=[
            pl.BlockSpec(memory_space=pltpu.SMEM),
            pl.BlockSpec(memory_space=pltpu.SMEM),
            pl.BlockSpec((1, _SEQ, _DIN), lambda i: (i, 0, 0)),
            pl.BlockSpec((_DIN, _DOUT), lambda i: (0, 0)),
            pl.BlockSpec((1, _DOUT), lambda i: (0, 0)),
            pl.BlockSpec((_E, _R, _DIN), lambda i: (0, 0, 0)),
            pl.BlockSpec((_E, _R, _DOUT), lambda i: (0, 0, 0)),
            pl.BlockSpec((_E, _DOUT), lambda i: (0, 0)),
        ],
        out_specs=pl.BlockSpec((1, _SEQ, _DOUT), lambda i: (i, 0, 0)),
        out_shape=jax.ShapeDtypeStruct((_BSZ, _SEQ, _DOUT), jnp.float32),
    )(idx, attn, x, wt, b2, apt, bp, bias_pool)


def kernel(x, topk_attn, topk_idx, W, b, A_pool, B_pool, bias_pool):
    wt = W.T
    apt = A_pool.transpose(0, 2, 1)
    b2 = b.reshape(1, _DOUT)
    idx = topk_idx.astype(jnp.int32)
    return _run(x, topk_attn, idx, wt, b2, apt, B_pool, bias_pool)
